# baseline (speedup 1.0000x reference)
import functools

import jax
import jax.numpy as jnp
from jax import lax
from jax.experimental import pallas as pl
from jax.experimental.pallas import tpu as pltpu
from jax.experimental.pallas import tpu_sc as plsc

NC = 2
NS = 16
LANES = 16
NW = NC * NS

B_SC = 1024
TB = 16

_GDN = lax.GatherDimensionNumbers(
    offset_dims=(), collapsed_slice_dims=(0,), start_index_map=(0,))


def _bcast_lane(v, lane):
    idx = jnp.full((LANES, 1), lane, jnp.int32)
    return lax.gather(v, idx, _GDN, (1,),
                      mode=lax.GatherScatterMode.PROMISE_IN_BOUNDS)


def _sc_part(a_sc, T2, finT, L, A, S, O):
    B_sc = a_sc.shape[0]
    BW = B_sc // NW
    CH = 2
    DI = 2
    NCH = BW // CH
    NBUF = 4
    NJB = S // LANES

    aWt = jnp.transpose(a_sc.reshape(NW, BW, L), (0, 2, 1))

    mesh = plsc.VectorSubcoreMesh(core_axis_name="c", subcore_axis_name="s")

    @functools.partial(
        pl.kernel,
        out_type=jax.ShapeDtypeStruct((B_sc, L * O), jnp.float32),
        mesh=mesh,
        scratch_types=[
            pltpu.VMEM((L, BW), jnp.int32),
            pltpu.VMEM((BW, S), jnp.float32),
            pltpu.VMEM((BW, L * O), jnp.float32),
            pltpu.VMEM((O, S), jnp.float32),
        ] + [pltpu.VMEM((CH, S * S), jnp.float32) for _ in range(NBUF)]
          + [pltpu.SemaphoreType.DMA for _ in range(NBUF)],
        compiler_params=pltpu.CompilerParams(needs_layout_passes=False),
    )
    def sc_k(a_hbm, t2_hbm, fin_hbm, out_hbm, a_v, s_v, out_v, fin_v,
             *bufsems):
        bufs = bufsems[:NBUF]
        sems = bufsems[NBUF:]
        w = lax.axis_index("s") * NC + lax.axis_index("c")

        pltpu.sync_copy(a_hbm.at[w], a_v)
        pltpu.sync_copy(fin_hbm, fin_v)

        iota16 = lax.iota(jnp.int32, LANES)
        e0row = jnp.where(iota16 == 0, 1.0, 0.0).astype(jnp.float32)
        zrow = jnp.zeros((LANES,), jnp.float32)

        def init_b(b, carry):
            s_v[b, pl.ds(0, LANES)] = e0row
            for jb in range(1, NJB):
                s_v[b, pl.ds(jb * LANES, LANES)] = zrow
            return carry

        lax.fori_loop(0, BW, init_b, 0)

        def issue(tt, cc, buf, sem):
            idx = a_v.at[tt, pl.ds(cc * CH, CH)]
            pltpu.async_copy(t2_hbm.at[idx], buf, sem)

        def wait(buf, sem):
            pltpu.make_async_copy(
                t2_hbm.at[a_v.at[0, pl.ds(0, CH)]], buf, sem).wait()

        for c in range(NBUF):
            issue(0, c, bufs[c], sems[c])

        def body_t(t, carry):
            for c in range(NCH):
                bi = c % NBUF
                buf, sem = bufs[bi], sems[bi]
                wait(buf, sem)

                def body_e(e, ecarry):
                    b = c * CH + e
                    bfull = jnp.full((LANES,), b, jnp.int32)
                    acc16 = [jnp.zeros((LANES,), jnp.float32)
                             for _ in range(4 * NJB)]

                    def body_io(io, accs):
                        accs = list(accs)
                        sv = s_v[b, pl.ds(io * LANES, LANES)]
                        for iu in range(LANES):
                            sb = _bcast_lane(sv, iu)
                            base = (io * LANES + iu) * S
                            h = (iu % 4) * NJB
                            for jb in range(NJB):
                                accs[h + jb] = accs[h + jb] + sb * buf[
                                    e, pl.ds(base + jb * LANES, LANES)]
                        return tuple(accs)

                    acc16 = lax.fori_loop(0, S // LANES, body_io,
                                          tuple(acc16))
                    acc = [acc16[jb] + acc16[NJB + jb]
                           + acc16[2 * NJB + jb] + acc16[3 * NJB + jb]
                           for jb in range(NJB)]

                    for jb in range(NJB):
                        s_v[b, pl.ds(jb * LANES, LANES)] = acc[jb]

                    outs = []
                    for o in range(O):
                        p = acc[0] * fin_v[o, pl.ds(0, LANES)]
                        for jb in range(1, NJB):
                            p = p + acc[jb] * fin_v[o, pl.ds(jb * LANES, LANES)]
                        outs.append(jnp.sum(p))
                    ovec = jnp.where(iota16 == 0, outs[0], outs[1])
                    col = t * O + (iota16 % O)
                    plsc.store_scatter(out_v, [bfull, col], ovec,
                                       mask=iota16 < O)
                    return ecarry

                lax.fori_loop(0, CH, body_e, 0)

                if c + NBUF < NCH:
                    issue(t, c + NBUF, buf, sem)
                else:
                    tnext = jnp.minimum(t + 1, L - 1)
                    issue(tnext, c + NBUF - NCH, buf, sem)
            return carry

        lax.fori_loop(0, L, body_t, 0)

        for c in range(NBUF):
            wait(bufs[c], sems[c])

        pltpu.sync_copy(out_v, out_hbm.at[pl.ds(w * BW, BW)])

    return sc_k(aWt, T2, finT).reshape(B_sc, L, O)


def _tc_part(a_tc, trans_prob, fin_matrix, L, A, S, O):
    N = a_tc.shape[0]
    NBLK = N // TB
    aT = a_tc.T

    def tc_body(aref, *refs):
        mats = refs[:TB]
        fin_ref = refs[TB]
        out_ref = refs[TB + 1]
        s_ref = refs[TB + 2]
        t = pl.program_id(0)
        j = pl.program_id(1)

        @pl.when(t == 0)
        def _init():
            col = lax.broadcasted_iota(jnp.int32, (TB, S), 1)
            s_ref[pl.ds(j * TB, TB), :] = jnp.where(
                col == 0, 1.0, 0.0).astype(jnp.float32)

        s = s_ref[pl.ds(j * TB, TB), :]
        M = jnp.concatenate([m[...] for m in mats], axis=0)
        ns = jnp.sum(s[:, :, None] * M, axis=1)
        s_ref[pl.ds(j * TB, TB), :] = ns
        out_ref[...] = jnp.sum(
            ns[:, :, None] * fin_ref[...][None, :, :], axis=1)[None]

    def mat_map(k):
        return lambda t, j, aref: (aref[t, j * TB + k], 0, 0)

    gspec = pltpu.PrefetchScalarGridSpec(
        num_scalar_prefetch=1,
        grid=(L, NBLK),
        in_specs=[pl.BlockSpec((1, S, S), mat_map(k)) for k in range(TB)]
                 + [pl.BlockSpec((S, O), lambda t, j, aref: (0, 0))],
        out_specs=pl.BlockSpec((1, TB, O), lambda t, j, aref: (t, j, 0)),
        scratch_shapes=[pltpu.VMEM((N, S), jnp.float32)],
    )
    out = pl.pallas_call(
        tc_body,
        grid_spec=gspec,
        out_shape=jax.ShapeDtypeStruct((L, N, O), jnp.float32),
    )(aT, *([trans_prob] * TB), fin_matrix)
    return jnp.transpose(out, (1, 0, 2))


def kernel(action_seq, trans_prob, fin_matrix):
    B, L = action_seq.shape
    A, S, _ = trans_prob.shape
    O = fin_matrix.shape[1]

    T2 = trans_prob.reshape(A, S * S)
    finT = fin_matrix.T

    out_sc = _sc_part(action_seq[:B_SC], T2, finT, L, A, S, O)
    if B_SC < B:
        out_tc = _tc_part(action_seq[B_SC:], trans_prob, fin_matrix,
                          L, A, S, O)
        return jnp.concatenate([out_sc, out_tc], axis=0)
    return out_sc


# speedup vs baseline: 1.0020x; 1.0020x over previous
import functools

import jax
import jax.numpy as jnp
from jax import lax
from jax.experimental import pallas as pl
from jax.experimental.pallas import tpu as pltpu
from jax.experimental.pallas import tpu_sc as plsc

NC = 2
NS = 16
LANES = 16
NW = NC * NS

B_SC = 1024
TB = 16

_GDN = lax.GatherDimensionNumbers(
    offset_dims=(), collapsed_slice_dims=(0,), start_index_map=(0,))


def _bcast_lane(v, lane):
    idx = jnp.full((LANES, 1), lane, jnp.int32)
    return lax.gather(v, idx, _GDN, (1,),
                      mode=lax.GatherScatterMode.PROMISE_IN_BOUNDS)


def _sc_part(a_sc, T2, finT, L, A, S, O):
    B_sc = a_sc.shape[0]
    BW = B_sc // NW
    CH = 2
    DI = 2
    NCH = BW // CH
    NBUF = 8
    NJB = S // LANES

    aWt = jnp.transpose(a_sc.reshape(NW, BW, L), (0, 2, 1))

    mesh = plsc.VectorSubcoreMesh(core_axis_name="c", subcore_axis_name="s")

    @functools.partial(
        pl.kernel,
        out_type=jax.ShapeDtypeStruct((B_sc, L * O), jnp.float32),
        mesh=mesh,
        scratch_types=[
            pltpu.VMEM((L, BW), jnp.int32),
            pltpu.VMEM((BW, S), jnp.float32),
            pltpu.VMEM((BW, L * O), jnp.float32),
            pltpu.VMEM((O, S), jnp.float32),
        ] + [pltpu.VMEM((CH, S * S), jnp.float32) for _ in range(NBUF)]
          + [pltpu.SemaphoreType.DMA for _ in range(NBUF)],
        compiler_params=pltpu.CompilerParams(needs_layout_passes=False),
    )
    def sc_k(a_hbm, t2_hbm, fin_hbm, out_hbm, a_v, s_v, out_v, fin_v,
             *bufsems):
        bufs = bufsems[:NBUF]
        sems = bufsems[NBUF:]
        w = lax.axis_index("s") * NC + lax.axis_index("c")

        pltpu.sync_copy(a_hbm.at[w], a_v)
        pltpu.sync_copy(fin_hbm, fin_v)

        iota16 = lax.iota(jnp.int32, LANES)
        e0row = jnp.where(iota16 == 0, 1.0, 0.0).astype(jnp.float32)
        zrow = jnp.zeros((LANES,), jnp.float32)

        def init_b(b, carry):
            s_v[b, pl.ds(0, LANES)] = e0row
            for jb in range(1, NJB):
                s_v[b, pl.ds(jb * LANES, LANES)] = zrow
            return carry

        lax.fori_loop(0, BW, init_b, 0)

        def issue(tt, cc, buf, sem):
            idx = a_v.at[tt, pl.ds(cc * CH, CH)]
            pltpu.async_copy(t2_hbm.at[idx], buf, sem)

        def wait(buf, sem):
            pltpu.make_async_copy(
                t2_hbm.at[a_v.at[0, pl.ds(0, CH)]], buf, sem).wait()

        for c in range(NBUF):
            issue(0, c, bufs[c], sems[c])

        def body_t(t, carry):
            for c in range(NCH):
                bi = c % NBUF
                buf, sem = bufs[bi], sems[bi]
                wait(buf, sem)

                def body_e(e, ecarry):
                    b = c * CH + e
                    bfull = jnp.full((LANES,), b, jnp.int32)
                    acc16 = [jnp.zeros((LANES,), jnp.float32)
                             for _ in range(4 * NJB)]

                    def body_io(io, accs):
                        accs = list(accs)
                        sv = s_v[b, pl.ds(io * LANES, LANES)]
                        for iu in range(LANES):
                            sb = _bcast_lane(sv, iu)
                            base = (io * LANES + iu) * S
                            h = (iu % 4) * NJB
                            for jb in range(NJB):
                                accs[h + jb] = accs[h + jb] + sb * buf[
                                    e, pl.ds(base + jb * LANES, LANES)]
                        return tuple(accs)

                    acc16 = lax.fori_loop(0, S // LANES, body_io,
                                          tuple(acc16))
                    acc = [acc16[jb] + acc16[NJB + jb]
                           + acc16[2 * NJB + jb] + acc16[3 * NJB + jb]
                           for jb in range(NJB)]

                    for jb in range(NJB):
                        s_v[b, pl.ds(jb * LANES, LANES)] = acc[jb]

                    outs = []
                    for o in range(O):
                        p = acc[0] * fin_v[o, pl.ds(0, LANES)]
                        for jb in range(1, NJB):
                            p = p + acc[jb] * fin_v[o, pl.ds(jb * LANES, LANES)]
                        outs.append(jnp.sum(p))
                    ovec = jnp.where(iota16 == 0, outs[0], outs[1])
                    col = t * O + (iota16 % O)
                    plsc.store_scatter(out_v, [bfull, col], ovec,
                                       mask=iota16 < O)
                    return ecarry

                lax.fori_loop(0, CH, body_e, 0)

                if c + NBUF < NCH:
                    issue(t, c + NBUF, buf, sem)
                else:
                    tnext = jnp.minimum(t + 1, L - 1)
                    issue(tnext, c + NBUF - NCH, buf, sem)
            return carry

        lax.fori_loop(0, L, body_t, 0)

        for c in range(NBUF):
            wait(bufs[c], sems[c])

        pltpu.sync_copy(out_v, out_hbm.at[pl.ds(w * BW, BW)])

    return sc_k(aWt, T2, finT).reshape(B_sc, L, O)


def _tc_part(a_tc, trans_prob, fin_matrix, L, A, S, O):
    N = a_tc.shape[0]
    NBLK = N // TB
    aT = a_tc.T

    def tc_body(aref, *refs):
        mats = refs[:TB]
        fin_ref = refs[TB]
        out_ref = refs[TB + 1]
        s_ref = refs[TB + 2]
        t = pl.program_id(0)
        j = pl.program_id(1)

        @pl.when(t == 0)
        def _init():
            col = lax.broadcasted_iota(jnp.int32, (TB, S), 1)
            s_ref[pl.ds(j * TB, TB), :] = jnp.where(
                col == 0, 1.0, 0.0).astype(jnp.float32)

        s = s_ref[pl.ds(j * TB, TB), :]
        M = jnp.concatenate([m[...] for m in mats], axis=0)
        ns = jnp.sum(s[:, :, None] * M, axis=1)
        s_ref[pl.ds(j * TB, TB), :] = ns
        out_ref[...] = jnp.sum(
            ns[:, :, None] * fin_ref[...][None, :, :], axis=1)[None]

    def mat_map(k):
        return lambda t, j, aref: (aref[t, j * TB + k], 0, 0)

    gspec = pltpu.PrefetchScalarGridSpec(
        num_scalar_prefetch=1,
        grid=(L, NBLK),
        in_specs=[pl.BlockSpec((1, S, S), mat_map(k)) for k in range(TB)]
                 + [pl.BlockSpec((S, O), lambda t, j, aref: (0, 0))],
        out_specs=pl.BlockSpec((1, TB, O), lambda t, j, aref: (t, j, 0)),
        scratch_shapes=[pltpu.VMEM((N, S), jnp.float32)],
    )
    out = pl.pallas_call(
        tc_body,
        grid_spec=gspec,
        out_shape=jax.ShapeDtypeStruct((L, N, O), jnp.float32),
    )(aT, *([trans_prob] * TB), fin_matrix)
    return jnp.transpose(out, (1, 0, 2))


def kernel(action_seq, trans_prob, fin_matrix):
    B, L = action_seq.shape
    A, S, _ = trans_prob.shape
    O = fin_matrix.shape[1]

    T2 = trans_prob.reshape(A, S * S)
    finT = fin_matrix.T

    out_sc = _sc_part(action_seq[:B_SC], T2, finT, L, A, S, O)
    if B_SC < B:
        out_tc = _tc_part(action_seq[B_SC:], trans_prob, fin_matrix,
                          L, A, S, O)
        return jnp.concatenate([out_sc, out_tc], axis=0)
    return out_sc


# speedup vs baseline: 1.0782x; 1.0760x over previous
import functools

import jax
import jax.numpy as jnp
from jax import lax
from jax.experimental import pallas as pl
from jax.experimental.pallas import tpu as pltpu
from jax.experimental.pallas import tpu_sc as plsc

NC = 2
NS = 16
LANES = 16
NW = NC * NS

_GDN = lax.GatherDimensionNumbers(
    offset_dims=(), collapsed_slice_dims=(0,), start_index_map=(0,))


def _bcast_lane(v, lane):
    idx = jnp.full((LANES, 1), lane, jnp.int32)
    return lax.gather(v, idx, _GDN, (1,),
                      mode=lax.GatherScatterMode.PROMISE_IN_BOUNDS)


def kernel(action_seq, trans_prob, fin_matrix):
    B, L = action_seq.shape
    A, S, _ = trans_prob.shape
    O = fin_matrix.shape[1]

    BW = B // NW
    CH = 4
    NCH = BW // CH
    NBUF = 4
    NJB = S // LANES
    HW = S * S // 2

    T2 = trans_prob.reshape(2 * A, HW)
    finT = fin_matrix.T

    aWt = jnp.transpose(action_seq.reshape(NW, BW, L), (0, 2, 1))
    a2 = jnp.stack([2 * aWt, 2 * aWt + 1], axis=2)

    mesh = plsc.VectorSubcoreMesh(core_axis_name="c", subcore_axis_name="s")

    @functools.partial(
        pl.kernel,
        out_type=jax.ShapeDtypeStruct((B, L * O), jnp.float32),
        mesh=mesh,
        scratch_types=[
            pltpu.VMEM((L, 2, BW), jnp.int32),
            pltpu.VMEM((BW, S), jnp.float32),
            pltpu.VMEM((BW, L * O), jnp.float32),
            pltpu.VMEM((O, S), jnp.float32),
        ] + [pltpu.VMEM((CH, HW), jnp.float32) for _ in range(2 * NBUF)]
          + [pltpu.SemaphoreType.DMA for _ in range(2 * NBUF)],
        compiler_params=pltpu.CompilerParams(needs_layout_passes=False),
    )
    def sc_k(a_hbm, t2_hbm, fin_hbm, out_hbm, a_v, s_v, out_v, fin_v,
             *bufsems):
        bufsL = bufsems[:NBUF]
        bufsR = bufsems[NBUF:2 * NBUF]
        semsL = bufsems[2 * NBUF:3 * NBUF]
        semsR = bufsems[3 * NBUF:]
        w = lax.axis_index("s") * NC + lax.axis_index("c")

        pltpu.sync_copy(a_hbm.at[w], a_v)
        pltpu.sync_copy(fin_hbm, fin_v)

        iota16 = lax.iota(jnp.int32, LANES)
        e0row = jnp.where(iota16 == 0, 1.0, 0.0).astype(jnp.float32)
        zrow = jnp.zeros((LANES,), jnp.float32)

        def init_b(b, carry):
            s_v[b, pl.ds(0, LANES)] = e0row
            for jb in range(1, NJB):
                s_v[b, pl.ds(jb * LANES, LANES)] = zrow
            return carry

        lax.fori_loop(0, BW, init_b, 0)

        def issue(tt, cc, bi):
            idxL = a_v.at[tt, 0, pl.ds(cc * CH, CH)]
            pltpu.async_copy(t2_hbm.at[idxL], bufsL[bi], semsL[bi])
            idxR = a_v.at[tt, 1, pl.ds(cc * CH, CH)]
            pltpu.async_copy(t2_hbm.at[idxR], bufsR[bi], semsR[bi])

        def wait(bi):
            pltpu.make_async_copy(
                t2_hbm.at[a_v.at[0, 0, pl.ds(0, CH)]],
                bufsL[bi], semsL[bi]).wait()
            pltpu.make_async_copy(
                t2_hbm.at[a_v.at[0, 1, pl.ds(0, CH)]],
                bufsR[bi], semsR[bi]).wait()

        for c in range(NBUF):
            issue(0, c, c)

        def body_t(t, carry):
            for c in range(NCH):
                bi = c % NBUF
                bufL, bufR = bufsL[bi], bufsR[bi]
                wait(bi)

                def body_e(e, ecarry):
                    b = c * CH + e
                    bfull = jnp.full((LANES,), b, jnp.int32)
                    acc16 = [jnp.zeros((LANES,), jnp.float32)
                             for _ in range(4 * NJB)]

                    def make_body(src, iobase):
                        def body_io(io, accs):
                            accs = list(accs)
                            sv = s_v[b, pl.ds((iobase + io) * LANES, LANES)]
                            for iu in range(LANES):
                                sb = _bcast_lane(sv, iu)
                                base = (io * LANES + iu) * S
                                h = (iu % 4) * NJB
                                for jb in range(NJB):
                                    accs[h + jb] = accs[h + jb] + sb * src[
                                        e, pl.ds(base + jb * LANES, LANES)]
                            return tuple(accs)
                        return body_io

                    acc16 = lax.fori_loop(0, S // LANES // 2,
                                          make_body(bufL, 0), tuple(acc16))
                    acc16 = lax.fori_loop(0, S // LANES // 2,
                                          make_body(bufR, S // LANES // 2),
                                          tuple(acc16))
                    acc = [acc16[jb] + acc16[NJB + jb]
                           + acc16[2 * NJB + jb] + acc16[3 * NJB + jb]
                           for jb in range(NJB)]

                    for jb in range(NJB):
                        s_v[b, pl.ds(jb * LANES, LANES)] = acc[jb]

                    outs = []
                    for o in range(O):
                        p = acc[0] * fin_v[o, pl.ds(0, LANES)]
                        for jb in range(1, NJB):
                            p = p + acc[jb] * fin_v[o, pl.ds(jb * LANES, LANES)]
                        outs.append(jnp.sum(p))
                    ovec = jnp.where(iota16 == 0, outs[0], outs[1])
                    col = t * O + (iota16 % O)
                    plsc.store_scatter(out_v, [bfull, col], ovec,
                                       mask=iota16 < O)
                    return ecarry

                lax.fori_loop(0, CH, body_e, 0)

                if c + NBUF < NCH:
                    issue(t, c + NBUF, bi)
                else:
                    tnext = jnp.minimum(t + 1, L - 1)
                    issue(tnext, c + NBUF - NCH, bi)
            return carry

        lax.fori_loop(0, L, body_t, 0)

        for c in range(NBUF):
            wait(c)

        pltpu.sync_copy(out_v, out_hbm.at[pl.ds(w * BW, BW)])

    return sc_k(a2, T2, finT).reshape(B, L, O)


# speedup vs baseline: 1.1851x; 1.0991x over previous
import functools

import jax
import jax.numpy as jnp
from jax import lax
from jax.experimental import pallas as pl
from jax.experimental.pallas import tpu as pltpu
from jax.experimental.pallas import tpu_sc as plsc

NC = 2
NS = 16
LANES = 16
NW = NC * NS

_GDN = lax.GatherDimensionNumbers(
    offset_dims=(), collapsed_slice_dims=(0,), start_index_map=(0,))


def _bcast_lane(v, lane):
    idx = jnp.full((LANES, 1), lane, jnp.int32)
    return lax.gather(v, idx, _GDN, (1,),
                      mode=lax.GatherScatterMode.PROMISE_IN_BOUNDS)


def kernel(action_seq, trans_prob, fin_matrix):
    B, L = action_seq.shape
    A, S, _ = trans_prob.shape
    O = fin_matrix.shape[1]

    BW = B // NW
    CH = 4
    DI = 2
    ND = CH // DI
    NCH = BW // CH
    NBUF = 4
    NJB = S // LANES

    T2 = trans_prob.reshape(A, S * S)
    finT = fin_matrix.T

    aWt = jnp.transpose(action_seq.reshape(NW, BW, L), (0, 2, 1))

    mesh = plsc.VectorSubcoreMesh(core_axis_name="c", subcore_axis_name="s")

    @functools.partial(
        pl.kernel,
        out_type=jax.ShapeDtypeStruct((B, L * O), jnp.float32),
        mesh=mesh,
        scratch_types=[
            pltpu.VMEM((L, BW), jnp.int32),
            pltpu.VMEM((BW, S), jnp.float32),
            pltpu.VMEM((BW, L * O), jnp.float32),
            pltpu.VMEM((O, S), jnp.float32),
        ] + [pltpu.VMEM((ND, DI, S * S), jnp.float32) for _ in range(NBUF)]
          + [pltpu.SemaphoreType.DMA for _ in range(NBUF)],
        compiler_params=pltpu.CompilerParams(needs_layout_passes=False),
    )
    def sc_k(a_hbm, t2_hbm, fin_hbm, out_hbm, a_v, s_v, out_v, fin_v,
             *bufsems):
        bufs = bufsems[:NBUF]
        sems = bufsems[NBUF:]
        w = lax.axis_index("s") * NC + lax.axis_index("c")

        pltpu.sync_copy(a_hbm.at[w], a_v)
        pltpu.sync_copy(fin_hbm, fin_v)

        iota16 = lax.iota(jnp.int32, LANES)
        e0row = jnp.where(iota16 == 0, 1.0, 0.0).astype(jnp.float32)
        zrow = jnp.zeros((LANES,), jnp.float32)

        def init_b(b, carry):
            s_v[b, pl.ds(0, LANES)] = e0row
            for jb in range(1, NJB):
                s_v[b, pl.ds(jb * LANES, LANES)] = zrow
            return carry

        lax.fori_loop(0, BW, init_b, 0)

        def issue(tt, cc, buf, sem):
            for d in range(ND):
                idx = a_v.at[tt, pl.ds(cc * CH + d * DI, DI)]
                pltpu.async_copy(t2_hbm.at[idx], buf.at[d], sem)

        def wait(buf, sem):
            pltpu.make_async_copy(
                t2_hbm.at[a_v.at[0, pl.ds(0, CH)]], buf, sem).wait()

        for c in range(NBUF):
            issue(0, c, bufs[c], sems[c])

        def body_t(t, carry):
            for c in range(NCH):
                bi = c % NBUF
                buf, sem = bufs[bi], sems[bi]
                wait(buf, sem)

                def body_e(e, ecarry):
                    b = c * CH + e
                    ed = e // DI
                    er = e % DI
                    bfull = jnp.full((LANES,), b, jnp.int32)
                    acc16 = [jnp.zeros((LANES,), jnp.float32)
                             for _ in range(4 * NJB)]

                    def body_io(io, accs):
                        accs = list(accs)
                        sv = s_v[b, pl.ds(io * LANES, LANES)]
                        for iu in range(LANES):
                            sb = _bcast_lane(sv, iu)
                            base = (io * LANES + iu) * S
                            h = (iu % 4) * NJB
                            for jb in range(NJB):
                                accs[h + jb] = accs[h + jb] + sb * buf[
                                    ed, er, pl.ds(base + jb * LANES, LANES)]
                        return tuple(accs)

                    acc16 = lax.fori_loop(0, S // LANES, body_io,
                                          tuple(acc16))
                    acc = [acc16[jb] + acc16[NJB + jb]
                           + acc16[2 * NJB + jb] + acc16[3 * NJB + jb]
                           for jb in range(NJB)]

                    for jb in range(NJB):
                        s_v[b, pl.ds(jb * LANES, LANES)] = acc[jb]

                    outs = []
                    for o in range(O):
                        p = acc[0] * fin_v[o, pl.ds(0, LANES)]
                        for jb in range(1, NJB):
                            p = p + acc[jb] * fin_v[o, pl.ds(jb * LANES, LANES)]
                        outs.append(jnp.sum(p))
                    ovec = jnp.where(iota16 == 0, outs[0], outs[1])
                    col = t * O + (iota16 % O)
                    plsc.store_scatter(out_v, [bfull, col], ovec,
                                       mask=iota16 < O)
                    return ecarry

                lax.fori_loop(0, CH, body_e, 0)

                if c + NBUF < NCH:
                    issue(t, c + NBUF, buf, sem)
                else:
                    tnext = jnp.minimum(t + 1, L - 1)
                    issue(tnext, c + NBUF - NCH, buf, sem)
            return carry

        lax.fori_loop(0, L, body_t, 0)

        for c in range(NBUF):
            wait(bufs[c], sems[c])

        pltpu.sync_copy(out_v, out_hbm.at[pl.ds(w * BW, BW)])

    return sc_k(aWt, T2, finT).reshape(B, L, O)


# speedup vs baseline: 1.2081x; 1.0194x over previous
import functools

import jax
import jax.numpy as jnp
from jax import lax
from jax.experimental import pallas as pl
from jax.experimental.pallas import tpu as pltpu
from jax.experimental.pallas import tpu_sc as plsc

NC = 2
NS = 16
LANES = 16
NW = NC * NS

_GDN = lax.GatherDimensionNumbers(
    offset_dims=(), collapsed_slice_dims=(0,), start_index_map=(0,))


def _bcast_lane(v, lane):
    idx = jnp.full((LANES, 1), lane, jnp.int32)
    return lax.gather(v, idx, _GDN, (1,),
                      mode=lax.GatherScatterMode.PROMISE_IN_BOUNDS)


def kernel(action_seq, trans_prob, fin_matrix):
    B, L = action_seq.shape
    A, S, _ = trans_prob.shape
    O = fin_matrix.shape[1]

    BW = B // NW
    CH = 4
    NCH = BW // CH
    NBUF = 4
    NJB = S // LANES

    T2 = trans_prob.reshape(A, S * S)
    finT = fin_matrix.T

    aWt = jnp.transpose(action_seq.reshape(NW, BW, L), (0, 2, 1))

    mesh = plsc.VectorSubcoreMesh(core_axis_name="c", subcore_axis_name="s")

    @functools.partial(
        pl.kernel,
        out_type=jax.ShapeDtypeStruct((B, L * O), jnp.float32),
        mesh=mesh,
        scratch_types=[
            pltpu.VMEM((L, BW), jnp.int32),
            pltpu.VMEM((BW, S), jnp.float32),
            pltpu.VMEM((BW, L * O), jnp.float32),
            pltpu.VMEM((O, S), jnp.float32),
        ] + [pltpu.VMEM((CH, S * S), jnp.float32) for _ in range(NBUF)]
          + [pltpu.SemaphoreType.DMA for _ in range(NBUF)],
        compiler_params=pltpu.CompilerParams(needs_layout_passes=False),
    )
    def sc_k(a_hbm, t2_hbm, fin_hbm, out_hbm, a_v, s_v, out_v, fin_v,
             *bufsems):
        bufs = bufsems[:NBUF]
        sems = bufsems[NBUF:]
        w = lax.axis_index("s") * NC + lax.axis_index("c")

        pltpu.sync_copy(a_hbm.at[w], a_v)
        pltpu.sync_copy(fin_hbm, fin_v)

        iota16 = lax.iota(jnp.int32, LANES)
        e0row = jnp.where(iota16 == 0, 1.0, 0.0).astype(jnp.float32)
        zrow = jnp.zeros((LANES,), jnp.float32)

        def init_b(b, carry):
            s_v[b, pl.ds(0, LANES)] = e0row
            for jb in range(1, NJB):
                s_v[b, pl.ds(jb * LANES, LANES)] = zrow
            return carry

        lax.fori_loop(0, BW, init_b, 0)

        def issue(tt, cc, buf, sem):
            idx = a_v.at[tt, pl.ds(cc * CH, CH)]
            pltpu.async_copy(t2_hbm.at[idx], buf, sem)

        def wait(buf, sem):
            pltpu.make_async_copy(
                t2_hbm.at[a_v.at[0, pl.ds(0, CH)]], buf, sem).wait()

        for c in range(NBUF):
            issue(0, c, bufs[c], sems[c])

        def body_t(t, carry):
            for c in range(NCH):
                bi = c % NBUF
                buf, sem = bufs[bi], sems[bi]
                wait(buf, sem)

                def body_e(e, ecarry):
                    b = c * CH + e
                    bfull = jnp.full((LANES,), b, jnp.int32)
                    acc16 = [jnp.zeros((LANES,), jnp.float32)
                             for _ in range(4 * NJB)]

                    def body_io(io, accs):
                        accs = list(accs)
                        sv = s_v[b, pl.ds(io * LANES, LANES)]
                        for iu in range(LANES):
                            sb = _bcast_lane(sv, iu)
                            base = (io * LANES + iu) * S
                            h = (iu % 4) * NJB
                            for jb in range(NJB):
                                accs[h + jb] = accs[h + jb] + sb * buf[
                                    e, pl.ds(base + jb * LANES, LANES)]
                        return tuple(accs)

                    acc16 = lax.fori_loop(0, S // LANES, body_io,
                                          tuple(acc16))
                    acc = [acc16[jb] + acc16[NJB + jb]
                           + acc16[2 * NJB + jb] + acc16[3 * NJB + jb]
                           for jb in range(NJB)]

                    for jb in range(NJB):
                        s_v[b, pl.ds(jb * LANES, LANES)] = acc[jb]

                    outs = []
                    for o in range(O):
                        p = acc[0] * fin_v[o, pl.ds(0, LANES)]
                        for jb in range(1, NJB):
                            p = p + acc[jb] * fin_v[o, pl.ds(jb * LANES, LANES)]
                        outs.append(jnp.sum(p))
                    ovec = jnp.where(iota16 == 0, outs[0], outs[1])
                    col = t * O + (iota16 % O)
                    plsc.store_scatter(out_v, [bfull, col], ovec,
                                       mask=iota16 < O)
                    return ecarry

                lax.fori_loop(0, CH, body_e, 0)

                if c + NBUF < NCH:
                    issue(t, c + NBUF, buf, sem)
                else:
                    tnext = jnp.minimum(t + 1, L - 1)
                    issue(tnext, c + NBUF - NCH, buf, sem)
            return carry

        lax.fori_loop(0, L, body_t, 0)

        for c in range(NBUF):
            wait(bufs[c], sems[c])

        pltpu.sync_copy(out_v, out_hbm.at[pl.ds(w * BW, BW)])

    return sc_k(aWt, T2, finT).reshape(B, L, O)
